# Initial kernel scaffold; baseline (speedup 1.0000x reference)
import functools

import jax
import jax.numpy as jnp
from jax import lax
from jax.experimental import pallas as pl
from jax.experimental.pallas import tpu as pltpu
from jax.experimental.pallas import tpu_sc as plsc

NC = 2
NS = 16
NW = NC * NS
CHUNK = 128
GP = 10
POS_ROWS = 800


def _gather_table(wid, idx_hbm, tab, out_hbm, idx_v, rows, sem, nch, per_w):
    pltpu.sync_copy(idx_hbm.at[wid], idx_v)
    grp_rows = GP * CHUNK

    def body(g, carry):
        cps = []
        for j in range(GP):
            cps.append(
                pltpu.async_copy(
                    tab.at[idx_v.at[g * GP + j]],
                    rows.at[pl.ds(j * CHUNK, CHUNK)],
                    sem,
                )
            )
        for cp in cps:
            cp.wait()
        base = wid * per_w + g * grp_rows
        pltpu.sync_copy(rows, out_hbm.at[pl.ds(base, grp_rows)])
        return carry

    lax.fori_loop(0, nch // GP, body, 0)


def kernel(qids, uids, vids, clicks, Q_table, U_table, C_table, V_table, P_table):
    B, L = qids.shape
    N = B * L
    per_w = N // NW
    nch = per_w // CHUNK
    E = Q_table.shape[1]
    CE = C_table.shape[1]

    qi = qids.reshape(NW, nch, CHUNK)
    ui = uids.reshape(NW, nch, CHUNK)
    vi = vids.reshape(NW, nch, CHUNK)
    ci = clicks.reshape(NW, nch, CHUNK)

    mesh = plsc.VectorSubcoreMesh(core_axis_name="c", subcore_axis_name="s")

    @functools.partial(
        pl.kernel,
        mesh=mesh,
        out_type=[
            jax.ShapeDtypeStruct((N, E), jnp.float32),
            jax.ShapeDtypeStruct((N, E), jnp.float32),
            jax.ShapeDtypeStruct((N, CE), jnp.float32),
            jax.ShapeDtypeStruct((N, CE), jnp.float32),
            jax.ShapeDtypeStruct((N, CE), jnp.float32),
        ],
        scratch_types=[
            pltpu.VMEM((nch, CHUNK), jnp.int32),
            pltpu.VMEM((GP * CHUNK, E), jnp.float32),
            pltpu.VMEM((GP * CHUNK, CE), jnp.float32),
            pltpu.VMEM((POS_ROWS, CE), jnp.float32),
            pltpu.SemaphoreType.DMA,
        ],
    )
    def k(qi_h, ui_h, vi_h, ci_h, Qt, Ut, Ct, Vt, Pt,
          oq, ou, oc, ov, opos, idx_v, r32, r16, posb, sem):
        wid = lax.axis_index("s") * NC + lax.axis_index("c")

        _gather_table(wid, qi_h, Qt, oq, idx_v, r32, sem, nch, per_w)
        _gather_table(wid, ui_h, Ut, ou, idx_v, r32, sem, nch, per_w)
        _gather_table(wid, ci_h, Ct, oc, idx_v, r16, sem, nch, per_w)
        _gather_table(wid, vi_h, Vt, ov, idx_v, r16, sem, nch, per_w)

        pltpu.sync_copy(Pt, posb.at[pl.ds(0, 10)])
        n = 10
        while n < POS_ROWS:
            m = min(n, POS_ROWS - n)
            pltpu.sync_copy(posb.at[pl.ds(0, m)], posb.at[pl.ds(n, m)])
            n += m

        def pos_body(t, carry):
            pltpu.sync_copy(
                posb, opos.at[pl.ds(wid * per_w + t * POS_ROWS, POS_ROWS)]
            )
            return carry

        lax.fori_loop(0, per_w // POS_ROWS, pos_body, 0)

    oq, ou, oc, ov, opos = k(qi, ui, vi, ci, Q_table, U_table, C_table, V_table, P_table)
    return (
        oq.reshape(B, L, E),
        ou.reshape(B, L, E),
        oc.reshape(B, L, CE),
        ov.reshape(B, L, CE),
        opos.reshape(B, L, CE),
    )


# speedup vs baseline: 1.4238x; 1.4238x over previous
import functools

import jax
import jax.numpy as jnp
from jax import lax
from jax.experimental import pallas as pl
from jax.experimental.pallas import tpu as pltpu
from jax.experimental.pallas import tpu_sc as plsc

NC = 2
NS = 16
NW = NC * NS
CHUNK = 128
GP = 10
POS_ROWS = 800


def _gather_table(wid, idx_hbm, tab, out_hbm, idx_v, rows, sem, nch, per_w):
    pltpu.sync_copy(idx_hbm.at[wid], idx_v)
    grp_rows = GP * CHUNK

    def body(g, carry):
        cps = []
        for j in range(GP):
            cps.append(
                pltpu.async_copy(
                    tab.at[idx_v.at[g * GP + j]],
                    rows.at[pl.ds(j * CHUNK, CHUNK)],
                    sem,
                )
            )
        for cp in cps:
            cp.wait()
        base = wid * per_w + g * grp_rows
        pltpu.sync_copy(rows, out_hbm.at[pl.ds(base, grp_rows)])
        return carry

    lax.fori_loop(0, nch // GP, body, 0)


def kernel(qids, uids, vids, clicks, Q_table, U_table, C_table, V_table, P_table):
    B, L = qids.shape
    N = B * L
    per_w = N // NW
    nch = per_w // CHUNK
    E = Q_table.shape[1]
    CE = C_table.shape[1]

    qi = qids.reshape(NW, nch, CHUNK)
    ui = uids.reshape(NW, nch, CHUNK)
    vi = vids.reshape(NW, nch, CHUNK)
    ci = clicks.reshape(NW, nch, CHUNK)

    mesh = plsc.VectorSubcoreMesh(core_axis_name="c", subcore_axis_name="s")

    @functools.partial(
        pl.kernel,
        mesh=mesh,
        compiler_params=pltpu.CompilerParams(use_tc_tiling_on_sc=False),
        out_type=[
            jax.ShapeDtypeStruct((N, E), jnp.float32),
            jax.ShapeDtypeStruct((N, E), jnp.float32),
            jax.ShapeDtypeStruct((N, CE), jnp.float32),
            jax.ShapeDtypeStruct((N, CE), jnp.float32),
            jax.ShapeDtypeStruct((N, CE), jnp.float32),
        ],
        scratch_types=[
            pltpu.VMEM((nch, CHUNK), jnp.int32),
            pltpu.VMEM((GP * CHUNK, E), jnp.float32),
            pltpu.VMEM((GP * CHUNK, CE), jnp.float32),
            pltpu.VMEM((POS_ROWS, CE), jnp.float32),
            pltpu.VMEM((10, CE), jnp.float32),
            pltpu.SemaphoreType.DMA,
        ],
    )
    def k(qi_h, ui_h, vi_h, ci_h, Qt, Ut, Ct, Vt, Pt,
          oq, ou, oc, ov, opos, idx_v, r32, r16, posb, pv, sem):
        wid = lax.axis_index("s") * NC + lax.axis_index("c")

        _gather_table(wid, qi_h, Qt, oq, idx_v, r32, sem, nch, per_w)
        _gather_table(wid, ui_h, Ut, ou, idx_v, r32, sem, nch, per_w)
        _gather_table(wid, ci_h, Ct, oc, idx_v, r16, sem, nch, per_w)
        _gather_table(wid, vi_h, Vt, ov, idx_v, r16, sem, nch, per_w)

        pltpu.sync_copy(Pt, pv)
        prow = [pv[i, :] for i in range(10)]
        for b in range(POS_ROWS // 10):
            for r in range(10):
                posb[b * 10 + r, :] = prow[r]

        def pos_body(t, carry):
            pltpu.sync_copy(
                posb, opos.at[pl.ds(wid * per_w + t * POS_ROWS, POS_ROWS)]
            )
            return carry

        lax.fori_loop(0, per_w // POS_ROWS, pos_body, 0)

    oq, ou, oc, ov, opos = k(qi, ui, vi, ci, Q_table, U_table, C_table, V_table, P_table)
    return (
        oq.reshape(B, L, E),
        ou.reshape(B, L, E),
        oc.reshape(B, L, CE),
        ov.reshape(B, L, CE),
        opos.reshape(B, L, CE),
    )


# speedup vs baseline: 1.4352x; 1.0080x over previous
import functools

import jax
import jax.numpy as jnp
from jax import lax
from jax.experimental import pallas as pl
from jax.experimental.pallas import tpu as pltpu
from jax.experimental.pallas import tpu_sc as plsc

NC = 2
NS = 16
NW = NC * NS
CHUNK = 128
GP = 8
POS_ROWS = 400
POS_Q = 8

GRP_ROWS = GP * CHUNK


def _do_table(wid, idx_hbm, tab, out_hbm, idx_v, rows, sem_g, sem_s, nch, per_w):
    ngrp = nch // GP
    pltpu.sync_copy(idx_hbm.at[wid], idx_v)

    def fire(g, h):
        for j in range(GP):
            pltpu.async_copy(
                tab.at[idx_v.at[g * GP + j]],
                rows.at[pl.ds(h * GRP_ROWS + j * CHUNK, CHUNK)],
                sem_g,
            )

    def wait_gathers(h):
        for j in range(GP):
            pltpu.make_async_copy(
                tab.at[idx_v.at[j]],
                rows.at[pl.ds(h * GRP_ROWS + j * CHUNK, CHUNK)],
                sem_g,
            ).wait()

    def store(g, h):
        base = wid * per_w + g * GRP_ROWS
        return pltpu.async_copy(
            rows.at[pl.ds(h * GRP_ROWS, GRP_ROWS)],
            out_hbm.at[pl.ds(base, GRP_ROWS)],
            sem_s,
        )

    def wait_store(g, h):
        base = wid * per_w + g * GRP_ROWS
        pltpu.make_async_copy(
            rows.at[pl.ds(h * GRP_ROWS, GRP_ROWS)],
            out_hbm.at[pl.ds(base, GRP_ROWS)],
            sem_s,
        ).wait()

    fire(0, 0)
    fire(1, 1)

    def body(g, carry):
        h = g % 2
        wait_gathers(h)
        store(g, h)
        wait_store(g, h)
        fire_g = g + 2
        for j in range(GP):
            pltpu.async_copy(
                tab.at[idx_v.at[fire_g * GP + j]],
                rows.at[pl.ds(h * GRP_ROWS + j * CHUNK, CHUNK)],
                sem_g,
            )
        return carry

    lax.fori_loop(0, ngrp - 2, body, 0)

    for g in (ngrp - 2, ngrp - 1):
        h = g % 2
        wait_gathers(h)
        store(g, h)
    for g in (ngrp - 2, ngrp - 1):
        wait_store(g, g % 2)


def kernel(qids, uids, vids, clicks, Q_table, U_table, C_table, V_table, P_table):
    B, L = qids.shape
    N = B * L
    per_w = N // NW
    nch = per_w // CHUNK
    E = Q_table.shape[1]
    CE = C_table.shape[1]

    qi = qids.reshape(NW, nch, CHUNK)
    ui = uids.reshape(NW, nch, CHUNK)
    vi = vids.reshape(NW, nch, CHUNK)
    ci = clicks.reshape(NW, nch, CHUNK)

    mesh = plsc.VectorSubcoreMesh(core_axis_name="c", subcore_axis_name="s")

    @functools.partial(
        pl.kernel,
        mesh=mesh,
        compiler_params=pltpu.CompilerParams(use_tc_tiling_on_sc=False),
        out_type=[
            jax.ShapeDtypeStruct((N, E), jnp.float32),
            jax.ShapeDtypeStruct((N, E), jnp.float32),
            jax.ShapeDtypeStruct((N, CE), jnp.float32),
            jax.ShapeDtypeStruct((N, CE), jnp.float32),
            jax.ShapeDtypeStruct((N, CE), jnp.float32),
        ],
        scratch_types=[
            pltpu.VMEM((nch, CHUNK), jnp.int32),
            pltpu.VMEM((2 * GRP_ROWS, E), jnp.float32),
            pltpu.VMEM((2 * GRP_ROWS, CE), jnp.float32),
            pltpu.VMEM((POS_ROWS, CE), jnp.float32),
            pltpu.VMEM((10, CE), jnp.float32),
            pltpu.SemaphoreType.DMA,
            pltpu.SemaphoreType.DMA,
        ],
    )
    def k(qi_h, ui_h, vi_h, ci_h, Qt, Ut, Ct, Vt, Pt,
          oq, ou, oc, ov, opos, idx_v, r32, r16, posb, pv, sem_g, sem_s):
        wid = lax.axis_index("s") * NC + lax.axis_index("c")

        _do_table(wid, qi_h, Qt, oq, idx_v, r32, sem_g, sem_s, nch, per_w)
        _do_table(wid, ui_h, Ut, ou, idx_v, r32, sem_g, sem_s, nch, per_w)
        _do_table(wid, ci_h, Ct, oc, idx_v, r16, sem_g, sem_s, nch, per_w)
        _do_table(wid, vi_h, Vt, ov, idx_v, r16, sem_g, sem_s, nch, per_w)

        pltpu.sync_copy(Pt, pv)
        prow = [pv[i, :] for i in range(10)]
        for b in range(POS_ROWS // 10):
            for r in range(10):
                posb[b * 10 + r, :] = prow[r]

        nstores = per_w // POS_ROWS

        def pos_body(t, carry):
            cps = []
            for u in range(POS_Q):
                base = wid * per_w + (t * POS_Q + u) * POS_ROWS
                cps.append(
                    pltpu.async_copy(
                        posb, opos.at[pl.ds(base, POS_ROWS)], sem_s
                    )
                )
            for cp in cps:
                cp.wait()
            return carry

        lax.fori_loop(0, nstores // POS_Q, pos_body, 0)

    oq, ou, oc, ov, opos = k(qi, ui, vi, ci, Q_table, U_table, C_table, V_table, P_table)
    return (
        oq.reshape(B, L, E),
        ou.reshape(B, L, E),
        oc.reshape(B, L, CE),
        ov.reshape(B, L, CE),
        opos.reshape(B, L, CE),
    )
